# Initial kernel scaffold; baseline (speedup 1.0000x reference)
import jax
import jax.numpy as jnp
from jax.experimental import pallas as pl


def kernel(x, edge_index, W1l, W1r, att1, b1, W2l, W2r, att2, b2, W3, att3_src, att3_dst, b3):
    raise NotImplementedError("write your pallas kernel here")


# speedup vs baseline: 1.0799x; 1.0799x over previous
import functools

import jax
import jax.numpy as jnp
from jax.experimental import pallas as pl

N = 10000
E = 160000


def _mm_body(a_ref, w_ref, o_ref):
    o_ref[...] = jnp.dot(a_ref[...], w_ref[...],
                         preferred_element_type=jnp.float32)


def _pallas_matmul(a, w, bm=400):
    M, K = a.shape
    _, C = w.shape
    return pl.pallas_call(
        _mm_body,
        grid=(M // bm,),
        in_specs=[
            pl.BlockSpec((bm, K), lambda i: (i, 0)),
            pl.BlockSpec((K, C), lambda i: (0, 0)),
        ],
        out_specs=pl.BlockSpec((bm, C), lambda i: (i, 0)),
        out_shape=jax.ShapeDtypeStruct((M, C), jnp.float32),
    )(a, w)


def _seg_softmax_agg(ex, xl, src, dst, n, h, c):
    denom = jax.ops.segment_sum(ex, dst, n)
    num = jax.ops.segment_sum(
        xl[src].reshape(-1, h, c) * ex[:, :, None], dst, n)
    out = num / (denom[:, :, None] + 1e-16)
    return out.reshape(n, h * c)


def _gatv2(x, src, dst, Wl, Wr, att, b, h, c):
    xl = _pallas_matmul(x, Wl)
    xr = _pallas_matmul(x, Wr)
    e = jax.nn.leaky_relu(
        xl[src].reshape(-1, h, c) + xr[dst].reshape(-1, h, c), 0.2)
    s = (e * att[None, :, :]).sum(-1)
    ex = jnp.exp(s)
    return jax.nn.relu(_seg_softmax_agg(ex, xl, src, dst, N, h, c) + b)


def _seg_softmax_ref(scores, seg, num_segments):
    m = jax.ops.segment_max(scores, seg, num_segments)
    m = jnp.where(jnp.isfinite(m), m, 0.0)
    ex = jnp.exp(scores - m[seg])
    denom = jax.ops.segment_sum(ex, seg, num_segments)
    return ex / (denom[seg] + 1e-16)


def kernel(x, edge_index, W1l, W1r, att1, b1, W2l, W2r, att2, b2,
           W3, att3_src, att3_dst, b3):
    ar = jnp.arange(N, dtype=edge_index.dtype)
    ei = jnp.concatenate([edge_index, jnp.stack([ar, ar])], axis=1)
    src, dst = ei[0], ei[1]

    def gatv2_layer(xx, Wl, Wr, att, b, h, c):
        xl = _pallas_matmul(xx, Wl).reshape(N, h, c)
        xr = _pallas_matmul(xx, Wr).reshape(N, h, c)
        e = jax.nn.leaky_relu(xl[src] + xr[dst], 0.2)
        ex = jnp.exp((e * att[None, :, :]).sum(-1))
        denom = jax.ops.segment_sum(ex, dst, N)
        out = jax.ops.segment_sum(xl[src] * ex[:, :, None], dst, N)
        return (out / (denom[:, :, None] + 1e-16)).reshape(N, h * c) + b

    h1 = jax.nn.relu(gatv2_layer(x, W1l, W1r, att1, b1, 8, 256))
    h2 = jax.nn.relu(gatv2_layer(h1, W2l, W2r, att2, b2, 8, 128))
    xsp = _pallas_matmul(h2, jnp.concatenate(
        [W3, jnp.zeros((1024, 64), jnp.float32)], axis=1))
    xs = xsp[:, :64].reshape(N, 1, 64)
    a_src = (xs * att3_src[None, :, :]).sum(-1)
    a_dst = (xs * att3_dst[None, :, :]).sum(-1)
    alpha = jax.nn.leaky_relu(a_src[src] + a_dst[dst], 0.2)
    alpha = _seg_softmax_ref(alpha, dst, N)
    out = jax.ops.segment_sum(xs[src] * alpha[:, :, None], dst, N)
    return jax.nn.relu(out.reshape(N, 64) + b3)


def _kernel_v0(x, edge_index, W1l, W1r, att1, b1, W2l, W2r, att2, b2,
               W3, att3_src, att3_dst, b3):
    ar = jnp.arange(N, dtype=edge_index.dtype)
    ei = jnp.concatenate([edge_index, jnp.stack([ar, ar])], axis=1)
    src, dst = ei[0], ei[1]

    h1 = _gatv2(x, src, dst, W1l, W1r, att1, b1, 8, 256)
    h2 = _gatv2(h1, src, dst, W2l, W2r, att2, b2, 8, 128)

    Wp = jnp.concatenate(
        [W3, W3 @ att3_src.T, W3 @ att3_dst.T, jnp.zeros((1024, 62), jnp.float32)],
        axis=1)
    f = _pallas_matmul(h2, Wp)
    xs = f[:, :64]
    a_s, a_d = f[:, 64], f[:, 65]
    s = jax.nn.leaky_relu(a_s[src] + a_d[dst], 0.2)
    ex = jnp.exp(s)[:, None]
    denom = jax.ops.segment_sum(ex, dst, N)
    num = jax.ops.segment_sum(xs[src] * ex, dst, N)
    out = num / (denom + 1e-16)
    return jax.nn.relu(out + b3)


# speedup vs baseline: 4.3667x; 4.0436x over previous
import jax
import jax.numpy as jnp
from jax import lax
from jax.experimental import pallas as pl
from jax.experimental.pallas import tpu as pltpu
from jax.experimental.pallas import tpu_sc as plsc

N = 10000
E = 160000
EP = E + N
EPP = EP + 16
NW = 32
NODE_CHUNK = 313
NP = 10240


def _mm_body(a_ref, w_ref, o_ref):
    o_ref[...] = jnp.dot(a_ref[...], w_ref[...],
                         preferred_element_type=jnp.float32)


def _pallas_matmul(a, w, bm=400):
    M, K = a.shape
    _, C = w.shape
    return pl.pallas_call(
        _mm_body,
        grid=(M // bm,),
        in_specs=[
            pl.BlockSpec((bm, K), lambda i: (i, 0)),
            pl.BlockSpec((K, C), lambda i: (0, 0)),
        ],
        out_specs=pl.BlockSpec((bm, C), lambda i: (i, 0)),
        out_shape=jax.ShapeDtypeStruct((M, C), jnp.float32),
    )(a, w)


def _lane_bcast(v, lane):
    idx = jnp.full((16,), lane, jnp.int32)
    return v.at[idx].get(mode="promise_in_bounds")


def _lane_extract(v, lane):
    return _lane_bcast(v, lane)[0]


def _allsum(v, lanes):
    for sh in (8, 4, 2, 1):
        v = v + v.at[lanes ^ sh].get(mode="promise_in_bounds")
    return v


def _make_gatv2_edge(D, H, C):
    CSL = C // 16
    NSL = D // 16
    mesh = plsc.VectorSubcoreMesh(core_axis_name="c", subcore_axis_name="s")

    def body(xl_h, xr_h, att_h, b_h, src_h, dst_h, starts_h, out_h,
             idx_v, dstc_v, rows_v, xr_v, att_v, b_v, acc_v, orow_v,
             starts_v, sem):
        w = lax.axis_index("s") * 2 + lax.axis_index("c")
        lanes = lax.iota(jnp.int32, 16)

        pltpu.sync_copy(starts_h, starts_v)
        pltpu.sync_copy(att_h, att_v)
        pltpu.sync_copy(b_h, b_v)

        sv = starts_v[pl.ds(w, 16)]
        e0 = sv[0]
        e1 = sv[1]
        lo = w * NODE_CHUNK

        def zero_acc():
            def zb(j, carry):
                acc_v[pl.ds(j * 16, 16)] = jnp.zeros((16,), jnp.float32)
                return carry
            lax.fori_loop(0, NSL, zb, 0)

        def finalize(d, denom):
            rden = 1.0 / (denom + 1e-16)
            for h in range(H):
                r_h = _lane_bcast(rden, h)
                for cc in range(CSL):
                    off = h * C + cc * 16
                    v = acc_v[pl.ds(off, 16)] * r_h + b_v[pl.ds(off, 16)]
                    orow_v[pl.ds(off, 16)] = jnp.maximum(v, 0.0)
            pltpu.sync_copy(orow_v, out_h.at[d])

        zero_acc()
        pltpu.sync_copy(xr_h.at[lo], xr_v)

        c0 = (e0 // 16) * 16
        nch = (e1 - c0 + 15) // 16

        def chunk_body(ci, carry):
            cb = c0 + ci * 16
            pltpu.sync_copy(src_h.at[pl.ds(cb, 16)], idx_v)
            pltpu.sync_copy(dst_h.at[pl.ds(cb, 32)], dstc_v)
            pltpu.async_copy(xl_h.at[idx_v], rows_v, sem).wait()
            dstv = dstc_v[pl.ds(0, 16)]

            def edge_body(i, carry2):
                d_cur, denom = carry2
                e = cb + i
                active = (e >= e0) & (e < e1)
                d_e = dstc_v[pl.ds(i, 16)][0]
                adv = active & (d_e != d_cur)

                @pl.when(adv)
                def _():
                    finalize(d_cur, denom)
                    pltpu.sync_copy(xr_h.at[d_e], xr_v)
                    zero_acc()

                d_cur = jnp.where(adv, d_e, d_cur)
                denom = jnp.where(adv, jnp.zeros_like(denom), denom)

                svec = jnp.zeros((16,), jnp.float32)
                for h in range(H):
                    accv = jnp.zeros((16,), jnp.float32)
                    for cc in range(CSL):
                        off = h * C + cc * 16
                        z = rows_v[i, pl.ds(off, 16)] + xr_v[pl.ds(off, 16)]
                        z = jnp.maximum(z, 0.2 * z)
                        accv = accv + z * att_v[pl.ds(off, 16)]
                    sv = _allsum(accv, lanes)
                    svec = jnp.where(lanes == h, sv, svec)
                exv = jnp.exp(svec) * jnp.where(active, jnp.ones((16,), jnp.float32), jnp.zeros((16,), jnp.float32))
                denom = denom + exv

                for h in range(H):
                    w_h = _lane_bcast(exv, h)
                    for cc in range(CSL):
                        off = h * C + cc * 16
                        plsc.addupdate(acc_v.at[pl.ds(off, 16)],
                                       rows_v[i, pl.ds(off, 16)] * w_h)
                return d_cur, denom

            return lax.fori_loop(0, 16, edge_body, carry)

        d_cur, denom = lax.fori_loop(
            0, nch, chunk_body, (lo, jnp.zeros((16,), jnp.float32)))
        finalize(d_cur, denom)

    kern = pl.kernel(
        body,
        out_type=jax.ShapeDtypeStruct((N, D), jnp.float32),
        mesh=mesh,
        scratch_types=[
            pltpu.VMEM((16,), jnp.int32),
            pltpu.VMEM((32,), jnp.int32),
            pltpu.VMEM((16, D), jnp.float32),
            pltpu.VMEM((D,), jnp.float32),
            pltpu.VMEM((D,), jnp.float32),
            pltpu.VMEM((D,), jnp.float32),
            pltpu.VMEM((D,), jnp.float32),
            pltpu.VMEM((D,), jnp.float32),
            pltpu.VMEM((48,), jnp.int32),
            pltpu.SemaphoreType.DMA,
        ],
    )
    return kern


def _make_gatconv_edge():
    D = 64
    DT = 128
    NSL = D // 16
    mesh = plsc.VectorSubcoreMesh(core_axis_name="c", subcore_axis_name="s")

    def body(xf_h, b_h, src_h, dst_h, starts_h, out_h,
             idx_v, dstc_v, rows_v, xr_v, b_v, acc_v, orow_v,
             starts_v, sem):
        w = lax.axis_index("s") * 2 + lax.axis_index("c")

        pltpu.sync_copy(starts_h, starts_v)
        pltpu.sync_copy(b_h, b_v)

        sv = starts_v[pl.ds(w, 16)]
        e0 = sv[0]
        e1 = sv[1]
        lo = w * NODE_CHUNK

        def zero_acc():
            for j in range(NSL):
                acc_v[pl.ds(j * 16, 16)] = jnp.zeros((16,), jnp.float32)

        def finalize(d, denom):
            rden = 1.0 / (denom + 1e-16)
            for j in range(NSL):
                v = acc_v[pl.ds(j * 16, 16)] * rden + b_v[pl.ds(j * 16, 16)]
                orow_v[pl.ds(j * 16, 16)] = jnp.maximum(v, 0.0)
            pltpu.sync_copy(orow_v, out_h.at[d])

        zero_acc()
        pltpu.sync_copy(xf_h.at[lo], xr_v)

        c0 = (e0 // 16) * 16
        nch = (e1 - c0 + 15) // 16

        def chunk_body(ci, carry):
            cb = c0 + ci * 16
            pltpu.sync_copy(src_h.at[pl.ds(cb, 16)], idx_v)
            pltpu.sync_copy(dst_h.at[pl.ds(cb, 32)], dstc_v)
            pltpu.async_copy(xf_h.at[idx_v], rows_v, sem).wait()

            def edge_body(i, carry2):
                d_cur, denom = carry2
                e = cb + i
                active = (e >= e0) & (e < e1)
                d_e = dstc_v[pl.ds(i, 16)][0]
                adv = active & (d_e != d_cur)

                @pl.when(adv)
                def _():
                    finalize(d_cur, denom)
                    pltpu.sync_copy(xf_h.at[d_e], xr_v)
                    zero_acc()

                d_cur = jnp.where(adv, d_e, d_cur)
                denom = jnp.where(adv, jnp.zeros_like(denom), denom)

                a_s = rows_v[i, pl.ds(64, 16)][0]
                a_d = xr_v[pl.ds(64, 16)][0]
                s = a_s + a_d
                s = jnp.maximum(s, 0.2 * s)
                exv = jnp.exp(jnp.full((16,), s, jnp.float32)) * jnp.where(
                    active, jnp.ones((16,), jnp.float32),
                    jnp.zeros((16,), jnp.float32))
                denom = denom + exv
                for j in range(NSL):
                    plsc.addupdate(acc_v.at[pl.ds(j * 16, 16)],
                                   rows_v[i, pl.ds(j * 16, 16)] * exv)
                return d_cur, denom

            return lax.fori_loop(0, 16, edge_body, carry)

        d_cur, denom = lax.fori_loop(
            0, nch, chunk_body, (lo, jnp.zeros((16,), jnp.float32)))
        finalize(d_cur, denom)

    kern = pl.kernel(
        body,
        out_type=jax.ShapeDtypeStruct((N, D), jnp.float32),
        mesh=mesh,
        scratch_types=[
            pltpu.VMEM((16,), jnp.int32),
            pltpu.VMEM((32,), jnp.int32),
            pltpu.VMEM((16, DT), jnp.float32),
            pltpu.VMEM((DT,), jnp.float32),
            pltpu.VMEM((D,), jnp.float32),
            pltpu.VMEM((D,), jnp.float32),
            pltpu.VMEM((D,), jnp.float32),
            pltpu.VMEM((48,), jnp.int32),
            pltpu.SemaphoreType.DMA,
        ],
    )
    return kern


def kernel(x, edge_index, W1l, W1r, att1, b1, W2l, W2r, att2, b2,
           W3, att3_src, att3_dst, b3):
    ar = jnp.arange(N, dtype=jnp.int32)
    ei = jnp.concatenate([edge_index.astype(jnp.int32),
                          jnp.stack([ar, ar])], axis=1)
    src, dst = ei[0], ei[1]
    perm = jnp.argsort(dst)
    src_s = jnp.concatenate([src[perm], jnp.zeros((EPP - EP,), jnp.int32)])
    dst_s = jnp.concatenate([dst[perm], jnp.full((EPP - EP,), N - 1, jnp.int32)])
    bounds = jnp.minimum(jnp.arange(NW + 1, dtype=jnp.int32) * NODE_CHUNK, N)
    starts = jnp.searchsorted(dst_s[:EP], bounds).astype(jnp.int32)
    starts_pad = jnp.concatenate([starts, jnp.zeros((15,), jnp.int32)])

    def gatv2(h, Wl, Wr, att, b, H, C):
        xl = _pallas_matmul(h, Wl)
        xr = _pallas_matmul(h, Wr)
        edge = _make_gatv2_edge(H * C, H, C)
        return edge(xl, xr, att.reshape(-1), b, src_s, dst_s, starts_pad)

    h1 = gatv2(x, W1l, W1r, att1, b1, 8, 256)
    h2 = gatv2(h1, W2l, W2r, att2, b2, 8, 128)

    W3p = jnp.pad(W3, ((0, 0), (0, 64)))
    xsp = _pallas_matmul(h2, W3p)
    A = jnp.zeros((128, 128), jnp.float32)
    A = A.at[:64, 0].set(att3_src[0]).at[:64, 1].set(att3_dst[0])
    av = _pallas_matmul(xsp, A)
    xf = jnp.concatenate(
        [xsp[:, :64], av[:, :2], jnp.zeros((N, 62), jnp.float32)], axis=1)
    edge3 = _make_gatconv_edge()
    return edge3(xf, b3, src_s, dst_s, starts_pad)
